# baseline (speedup 1.0000x reference)
import jax
import jax.numpy as jnp
from jax import lax
from jax.experimental import pallas as pl
from jax.experimental.pallas import tpu as pltpu
from jax.experimental.pallas import tpu_sc as plsc

_NOISE = 0.001
_SCALE = -1.0 / _NOISE

_NB = 4
_W = 512
_ROWS = 16 * 512
_NPER = _ROWS * _W

_R_SC = 2048
_R_TC = _ROWS - _R_SC
_TC_CHUNK = 1024

_NW = 32
_PERC = _R_SC // 2
_SCH = 256
_NCH = _PERC // _SCH
_L = 16
_U = 8
_TR = _SCH // 16


def _tc_body(x_ref, b_ref, o_ref):
    b = pl.program_id(0)
    t = pl.program_id(1)
    xv = x_ref[...]
    bv = b_ref[...]
    d = jnp.where(jnp.isfinite(bv), bv - xv, 0.0)
    s = _SCALE * jnp.sum(d * d)

    @pl.when(t == 0)
    def _init():
        o_ref[b] = s

    @pl.when(t != 0)
    def _acc():
        o_ref[b] += s


def _tile_sum(xt, bt, acc):
    gper = _W // (_L * _U)
    ngrp = _TR * gper

    def vec_body(g, accs):
        r = g // gper
        c0 = (g % gper) * _L * _U
        new = []
        for u in range(_U):
            xv = xt[r, pl.ds(c0 + u * _L, _L)]
            bv = bt[r, pl.ds(c0 + u * _L, _L)]
            m = jnp.abs(bv) < jnp.float32(jnp.inf)
            d = jnp.where(m, bv - xv, jnp.float32(0.0))
            new.append(accs[u] + d * d)
        return tuple(new)

    return plsc.parallel_loop(0, ngrp, unroll=2, carry=acc)(vec_body)


def _sc_body(x_hbm, b_hbm, out_hbm, xsh, bsh, xt, bt, pbuf, s0, s1, s2, s3):
    cid = lax.axis_index("c")
    sid = lax.axis_index("s")
    wid = sid * 2 + cid
    sems = ((s0, s1), (s2, s3))

    sched = [(b, k) for b in range(_NB) for k in range(_NCH)]
    ntot = len(sched)

    def make(ci):
        b, k = sched[ci]
        par = ci % 2
        sx, sb = sems[par]
        grp = (_R_TC + k * _SCH) // 16 + cid * (_PERC // 16)
        hx = pltpu.make_async_copy(
            x_hbm.at[b, pl.ds(grp, _SCH // 16)], xsh.at[par], sx)
        hb = pltpu.make_async_copy(
            b_hbm.at[b, pl.ds(grp, _SCH // 16)], bsh.at[par], sb)
        return hx, hb

    descs = [make(ci) for ci in range(ntot)]

    @pl.when(sid == 0)
    def _prime():
        for ci in (0, 1):
            descs[ci][0].start()
            descs[ci][1].start()

    accs = [tuple(jnp.zeros((_L,), jnp.float32) for _ in range(_U))
            for _ in range(_NB)]
    for ci in range(ntot):
        b = sched[ci][0]
        par = ci % 2

        @pl.when(sid == 0)
        def _wait():
            descs[ci][0].wait()
            descs[ci][1].wait()

        plsc.subcore_barrier()
        pltpu.sync_copy(xsh.at[par, sid], xt)
        pltpu.sync_copy(bsh.at[par, sid], bt)
        plsc.subcore_barrier()

        if ci + 2 < ntot:
            @pl.when(sid == 0)
            def _next():
                descs[ci + 2][0].start()
                descs[ci + 2][1].start()

        accs[b] = _tile_sum(xt, bt, accs[b])

    for b in range(_NB):
        a = accs[b]
        total = ((a[0] + a[1]) + (a[2] + a[3])) + (
            (a[4] + a[5]) + (a[6] + a[7]))
        pbuf[...] = total
        pltpu.sync_copy(pbuf, out_hbm.at[b, wid])


def kernel(x, batch):
    x2 = x.reshape(_NB, _ROWS, _W)
    b2 = batch.reshape(_NB, _ROWS, _W)

    tc_out = pl.pallas_call(
        _tc_body,
        grid=(_NB, _R_TC // _TC_CHUNK),
        in_specs=[
            pl.BlockSpec((1, _TC_CHUNK, _W), lambda b, t: (b, t, 0)),
            pl.BlockSpec((1, _TC_CHUNK, _W), lambda b, t: (b, t, 0)),
        ],
        out_specs=pl.BlockSpec(
            (_NB,), lambda b, t: (0,), memory_space=pltpu.SMEM
        ),
        out_shape=jax.ShapeDtypeStruct((_NB,), jnp.float32),
    )(x2, b2)

    mesh = plsc.VectorSubcoreMesh(core_axis_name="c", subcore_axis_name="s")
    sc_partial = pl.kernel(
        _sc_body,
        mesh=mesh,
        out_type=jax.ShapeDtypeStruct((_NB, _NW, _L), jnp.float32),
        scratch_types=[
            pltpu.VMEM_SHARED((2, _SCH // 16, _TR, _W), jnp.float32),
            pltpu.VMEM_SHARED((2, _SCH // 16, _TR, _W), jnp.float32),
            pltpu.VMEM((_TR, _W), jnp.float32),
            pltpu.VMEM((_TR, _W), jnp.float32),
            pltpu.VMEM((_L,), jnp.float32),
            pltpu.SemaphoreType.DMA,
            pltpu.SemaphoreType.DMA,
            pltpu.SemaphoreType.DMA,
            pltpu.SemaphoreType.DMA,
        ],
        compiler_params=pltpu.CompilerParams(use_tc_tiling_on_sc=True),
    )(x.reshape(_NB, _ROWS // 16, 16, _W), batch.reshape(_NB, _ROWS // 16, 16, _W))

    return tc_out + _SCALE * jnp.sum(sc_partial, axis=(1, 2))


# speedup vs baseline: 1.4897x; 1.4897x over previous
import jax
import jax.numpy as jnp
from jax.experimental import pallas as pl
from jax.experimental.pallas import tpu as pltpu

_NOISE = 0.001
_SCALE = -1.0 / _NOISE


def _nll_kernel(x_ref, b_ref, o_ref):
    b = pl.program_id(0)
    t = pl.program_id(1)
    xv = x_ref[...]
    bv = b_ref[...]
    d = jnp.where(jnp.isfinite(bv), bv - xv, 0.0)
    s = _SCALE * jnp.sum(d * d)

    @pl.when(t == 0)
    def _init():
        o_ref[b] = s

    @pl.when(t != 0)
    def _acc():
        o_ref[b] += s


def kernel(x, batch):
    nb, nt, h, w = x.shape
    x2 = x.reshape(nb, nt * h, w)
    b2 = batch.reshape(nb, nt * h, w)
    chunk = 2048
    n_chunks = (nt * h) // chunk

    out = pl.pallas_call(
        _nll_kernel,
        grid=(nb, n_chunks),
        in_specs=[
            pl.BlockSpec((1, chunk, w), lambda b, t: (b, t, 0)),
            pl.BlockSpec((1, chunk, w), lambda b, t: (b, t, 0)),
        ],
        out_specs=pl.BlockSpec(
            (nb,), lambda b, t: (0,), memory_space=pltpu.SMEM
        ),
        out_shape=jax.ShapeDtypeStruct((nb,), jnp.float32),
    )(x2, b2)
    return out
